# baseline (speedup 1.0000x reference)
import functools

import jax
import jax.numpy as jnp
from jax import lax
from jax.experimental import pallas as pl
from jax.experimental.pallas import tpu as pltpu
from jax.experimental.pallas import tpu_sc as plsc

N = 10000
H = 128
E = 320000
L = 3

NPAD = 10240
EPAD = 327680
CHUNK = 128
NROW = EPAD // CHUNK
NTILE = 16
NCORE = 2
ROWS_W = NROW // (NTILE * NCORE)
QROWS = 16
ZROWS = NPAD // NTILE

@functools.lru_cache(maxsize=None)
def _mesh():
    return plsc.VectorSubcoreMesh(core_axis_name="c", subcore_axis_name="s")


def _memset2d(ref, nrow):
    z16 = jnp.zeros((16,), jnp.float32)

    def body(k, carry):
        ref[k // 8, pl.ds((k % 8) * 16, 16)] = z16
        return carry

    lax.fori_loop(0, nrow * 8, body, 0)


def _memset1d(ref, n):
    z16 = jnp.zeros((16,), jnp.float32)

    def body(k, carry):
        ref[pl.ds(k * 16, 16)] = z16
        return carry

    lax.fori_loop(0, n // 16, body, 0)


@functools.lru_cache(maxsize=None)
def _make_sc_degree():
    return functools.partial(
        pl.kernel,
        mesh=_mesh(),
        out_type=jax.ShapeDtypeStruct((NCORE * 3 * NPAD,), jnp.float32),
        scratch_types=[
            pltpu.VMEM((ROWS_W, CHUNK), jnp.int32),
            pltpu.VMEM((CHUNK,), jnp.float32),
            pltpu.VMEM((ZROWS,), jnp.float32),
            pltpu.VMEM_SHARED((NPAD,), jnp.float32),
            pltpu.VMEM_SHARED((NPAD,), jnp.float32),
            pltpu.VMEM_SHARED((NPAD,), jnp.float32),
        ],
    )(_sc_degree_body)


def _sc_degree_body(dst_hbm, out_hbm, dst_v, ones_v, zvec_v, acc0, acc1, acc2):
    c = lax.axis_index("c")
    s = lax.axis_index("s")
    accs = (acc0, acc1, acc2)

    def o16(k, carry):
        ones_v[pl.ds(k * 16, 16)] = jnp.ones((16,), jnp.float32)
        return carry

    lax.fori_loop(0, CHUNK // 16, o16, 0)
    _memset1d(zvec_v, ZROWS)
    for t in range(3):
        pltpu.sync_copy(zvec_v, accs[t].at[pl.ds(s * ZROWS, ZROWS)])
    plsc.subcore_barrier()

    base = c * (NROW // 2) + s * ROWS_W
    for t in range(3):
        pltpu.sync_copy(dst_hbm.at[t].at[pl.ds(base, ROWS_W)], dst_v)

        def body(j, carry):
            pltpu.sync_copy(ones_v, accs[t].at[dst_v.at[j]], add=True)
            return carry

        lax.fori_loop(0, ROWS_W, body, 0)
    plsc.subcore_barrier()
    for t in range(3):
        off = (c * 3 + t) * NPAD + s * ZROWS
        pltpu.sync_copy(accs[t].at[pl.ds(s * ZROWS, ZROWS)],
                        out_hbm.at[pl.ds(off, ZROWS)])


@functools.lru_cache(maxsize=None)
def _make_sc_scatter():
    return functools.partial(
        pl.kernel,
        mesh=_mesh(),
        out_type=jax.ShapeDtypeStruct((NCORE, NPAD, H), jnp.float32),
        scratch_types=[
            pltpu.VMEM((QROWS, CHUNK), jnp.int32),
            pltpu.VMEM((QROWS, CHUNK), jnp.int32),
            pltpu.VMEM((CHUNK, H), jnp.float32),
            pltpu.VMEM((CHUNK, H), jnp.float32),
            pltpu.VMEM_SHARED((NPAD, H), jnp.float32),
            pltpu.SemaphoreType.DMA,
            pltpu.SemaphoreType.DMA,
            pltpu.SemaphoreType.DMA,
            pltpu.SemaphoreType.DMA,
        ],
    )(_sc_scatter_body)


def _sc_scatter_body(y_hbm, src_hbm, dst_hbm, out_hbm, src_v, dst_v, rows0,
                     rows1, acc, gsem0, gsem1, ssem0, ssem1):
    c = lax.axis_index("c")
    s = lax.axis_index("s")
    bufs = (rows0, rows1)
    gsems = (gsem0, gsem1)
    ssems = (ssem0, ssem1)

    _memset2d(rows0, CHUNK)
    for b in range(ZROWS // CHUNK):
        pltpu.sync_copy(rows0, acc.at[pl.ds(s * ZROWS + b * CHUNK, CHUNK)])
    plsc.subcore_barrier()

    def gwait(b):
        pltpu.make_async_copy(y_hbm.at[pl.ds(0, CHUNK)], bufs[b],
                              gsems[b]).wait()

    def swait(b):
        pltpu.make_async_copy(y_hbm.at[pl.ds(0, CHUNK)], bufs[b],
                              ssems[b]).wait()

    base = c * (NROW // 2) + s * ROWS_W
    for q in range(ROWS_W // QROWS):
        pltpu.sync_copy(src_hbm.at[pl.ds(base + q * QROWS, QROWS)], src_v)
        pltpu.sync_copy(dst_hbm.at[pl.ds(base + q * QROWS, QROWS)], dst_v)
        pltpu.async_copy(y_hbm.at[src_v.at[0]], rows0, gsem0)

        def pair(g, carry):
            for b in range(2):
                j = g * 2 + b
                gwait(b)
                pltpu.async_copy(bufs[b], acc.at[dst_v.at[j]], ssems[b],
                                 add=True)

                @pl.when(j >= 1)
                def _():
                    swait(1 - b)

                @pl.when(j + 1 < QROWS)
                def _():
                    pltpu.async_copy(y_hbm.at[src_v.at[j + 1]], bufs[1 - b],
                                     gsems[1 - b])

            return carry

        lax.fori_loop(0, QROWS // 2, pair, 0)
        swait(1)
    plsc.subcore_barrier()
    pltpu.sync_copy(acc.at[pl.ds(s * ZROWS, ZROWS)],
                    out_hbm.at[c].at[pl.ds(s * ZROWS, ZROWS)])


def _mm_body(h_ref, w_ref, cnt_ref, ya_ref, yc_ref, yd_ref):
    xw = jnp.dot(h_ref[...], w_ref[...], preferred_element_type=jnp.float32)
    cnt = cnt_ref[...]
    dinv = lax.rsqrt(cnt[0] + cnt[1] + 1.0)
    ya_ref[...] = xw[:, 0 * H:1 * H] * dinv[0][:, None]
    yc_ref[...] = xw[:, 1 * H:2 * H] * dinv[1][:, None]
    yd_ref[...] = xw[:, 2 * H:3 * H] * dinv[2][:, None]


def _tc_matmul(h, wcat, cnt):
    blk = 1024
    g = NPAD // blk
    return pl.pallas_call(
        _mm_body,
        grid=(g,),
        in_specs=[
            pl.BlockSpec((blk, H), lambda i: (i, 0)),
            pl.BlockSpec((H, 3 * H), lambda i: (0, 0)),
            pl.BlockSpec((NCORE, 3, blk), lambda i: (0, 0, i)),
        ],
        out_specs=[pl.BlockSpec((blk, H), lambda i: (i, 0))] * 3,
        out_shape=[jax.ShapeDtypeStruct((NPAD, H), jnp.float32)] * 3,
    )(h, wcat, cnt)


def _gru_body(sa_ref, sc_ref, sd_ref, ya_ref, yc_ref, yd_ref, h_ref, cnt_ref,
              bcat_ref, wih_ref, whh_ref, bih_ref, bhh_ref, out_ref):
    cnt = cnt_ref[...]
    dinv = lax.rsqrt(cnt[0] + cnt[1] + 1.0)
    bcat = bcat_ref[...]
    a = jnp.zeros_like(h_ref[...])
    for t, (s_ref, y_ref) in enumerate(((sa_ref, ya_ref), (sc_ref, yc_ref),
                                        (sd_ref, yd_ref))):
        st = s_ref[...]
        tot = st[0] + st[1] + y_ref[...]
        a = a + tot * dinv[t][:, None] + bcat[t][None, :]
    h = h_ref[...]
    gi = jnp.dot(a, wih_ref[...], preferred_element_type=jnp.float32) + bih_ref[...]
    gh = jnp.dot(h, whh_ref[...], preferred_element_type=jnp.float32) + bhh_ref[...]
    r = 1.0 / (1.0 + jnp.exp(-(gi[:, 0 * H:1 * H] + gh[:, 0 * H:1 * H])))
    z = 1.0 / (1.0 + jnp.exp(-(gi[:, 1 * H:2 * H] + gh[:, 1 * H:2 * H])))
    n = jnp.tanh(gi[:, 2 * H:3 * H] + r * gh[:, 2 * H:3 * H])
    out_ref[...] = (1.0 - z) * n + z * h


def _tc_gru(sa, sc, sd, ya, yc, yd, h, cnt, bcat, wihT, whhT, bih, bhh):
    blk = 1024
    g = NPAD // blk
    sspec = pl.BlockSpec((NCORE, blk, H), lambda i: (0, i, 0))
    yspec = pl.BlockSpec((blk, H), lambda i: (i, 0))
    return pl.pallas_call(
        _gru_body,
        grid=(g,),
        in_specs=[
            sspec, sspec, sspec, yspec, yspec, yspec, yspec,
            pl.BlockSpec((NCORE, 3, blk), lambda i: (0, 0, i)),
            pl.BlockSpec((3, H), lambda i: (0, 0)),
            pl.BlockSpec((H, 3 * H), lambda i: (0, 0)),
            pl.BlockSpec((H, 3 * H), lambda i: (0, 0)),
            pl.BlockSpec((1, 3 * H), lambda i: (0, 0)),
            pl.BlockSpec((1, 3 * H), lambda i: (0, 0)),
        ],
        out_specs=pl.BlockSpec((blk, H), lambda i: (i, 0)),
        out_shape=jax.ShapeDtypeStruct((NPAD, H), jnp.float32),
    )(sa, sc, sd, ya, yc, yd, h, cnt, bcat, wihT, whhT, bih, bhh)


def _prep_edges(e):
    e = e.astype(jnp.int32)
    pad_src = jnp.full((EPAD - E,), N, jnp.int32)
    pad_dst = N + jnp.arange(EPAD - E, dtype=jnp.int32) % (NPAD - N)
    src = jnp.concatenate([e[0], pad_src]).reshape(NROW, CHUNK)
    dst = jnp.concatenate([e[1], pad_dst]).reshape(NROW, CHUNK)
    return src, dst


def kernel(x, edge_ast, edge_cfg, edge_dfg, W_ast, b_ast, W_cfg, b_cfg,
           W_dfg, b_dfg, W_ih, W_hh, b_ih, b_hh):
    src_a, dst_a = _prep_edges(edge_ast)
    src_c, dst_c = _prep_edges(edge_cfg)
    src_d, dst_d = _prep_edges(edge_dfg)
    dst3 = jnp.stack([dst_a, dst_c, dst_d])

    h = jnp.zeros((NPAD, H), jnp.float32).at[:N].set(x)
    wcat = jnp.concatenate([W_ast, W_cfg, W_dfg], axis=1)
    bcat = jnp.stack([b_ast, b_cfg, b_dfg])
    wihT = W_ih.T
    whhT = W_hh.T
    bih = b_ih.reshape(1, 3 * H)
    bhh = b_hh.reshape(1, 3 * H)

    cnt = _make_sc_degree()(dst3).reshape(NCORE, 3, NPAD)

    scatter = _make_sc_scatter()
    for _ in range(L):
        ya, yc, yd = _tc_matmul(h, wcat, cnt)
        sa = scatter(ya, src_a, dst_a)
        sc = scatter(yc, src_c, dst_c)
        sd = scatter(yd, src_d, dst_d)
        h = _tc_gru(sa, sc, sd, ya, yc, yd, h, cnt, bcat, wihT, whhT, bih, bhh)
    return h[:N]


# speedup vs baseline: 1.0392x; 1.0392x over previous
import functools

import jax
import jax.numpy as jnp
from jax import lax
from jax.experimental import pallas as pl
from jax.experimental.pallas import tpu as pltpu
from jax.experimental.pallas import tpu_sc as plsc

N = 10000
H = 128
E = 320000
L = 3

NPAD = 10240
EPAD = 327680
CHUNK = 128
NROW = EPAD // CHUNK
NTILE = 16
NCORE = 2
ROWS_W = NROW // (NTILE * NCORE)
QROWS = 16
ZROWS = NPAD // NTILE

@functools.lru_cache(maxsize=None)
def _mesh():
    return plsc.VectorSubcoreMesh(core_axis_name="c", subcore_axis_name="s")


def _memset2d(ref, nrow):
    z16 = jnp.zeros((16,), jnp.float32)

    def body(k, carry):
        ref[k // 8, pl.ds((k % 8) * 16, 16)] = z16
        return carry

    lax.fori_loop(0, nrow * 8, body, 0)


def _memset1d(ref, n):
    z16 = jnp.zeros((16,), jnp.float32)

    def body(k, carry):
        ref[pl.ds(k * 16, 16)] = z16
        return carry

    lax.fori_loop(0, n // 16, body, 0)


@functools.lru_cache(maxsize=None)
def _make_sc_degree():
    return functools.partial(
        pl.kernel,
        mesh=_mesh(),
        out_type=jax.ShapeDtypeStruct((NCORE * 3 * NPAD,), jnp.float32),
        scratch_types=[
            pltpu.VMEM((ROWS_W, CHUNK), jnp.int32),
            pltpu.VMEM((CHUNK,), jnp.float32),
            pltpu.VMEM((ZROWS,), jnp.float32),
            pltpu.VMEM_SHARED((NPAD,), jnp.float32),
            pltpu.VMEM_SHARED((NPAD,), jnp.float32),
            pltpu.VMEM_SHARED((NPAD,), jnp.float32),
        ],
    )(_sc_degree_body)


def _sc_degree_body(dst_hbm, out_hbm, dst_v, ones_v, zvec_v, acc0, acc1, acc2):
    c = lax.axis_index("c")
    s = lax.axis_index("s")
    accs = (acc0, acc1, acc2)

    def o16(k, carry):
        ones_v[pl.ds(k * 16, 16)] = jnp.ones((16,), jnp.float32)
        return carry

    lax.fori_loop(0, CHUNK // 16, o16, 0)
    _memset1d(zvec_v, ZROWS)
    for t in range(3):
        pltpu.sync_copy(zvec_v, accs[t].at[pl.ds(s * ZROWS, ZROWS)])
    plsc.subcore_barrier()

    base = c * (NROW // 2) + s * ROWS_W
    for t in range(3):
        pltpu.sync_copy(dst_hbm.at[t].at[pl.ds(base, ROWS_W)], dst_v)

        def body(j, carry):
            pltpu.sync_copy(ones_v, accs[t].at[dst_v.at[j]], add=True)
            return carry

        lax.fori_loop(0, ROWS_W, body, 0)
    plsc.subcore_barrier()
    for t in range(3):
        off = (c * 3 + t) * NPAD + s * ZROWS
        pltpu.sync_copy(accs[t].at[pl.ds(s * ZROWS, ZROWS)],
                        out_hbm.at[pl.ds(off, ZROWS)])


@functools.lru_cache(maxsize=None)
def _make_sc_scatter():
    return functools.partial(
        pl.kernel,
        mesh=_mesh(),
        out_type=[jax.ShapeDtypeStruct((NCORE, NPAD, H), jnp.float32)] * 3,
        scratch_types=[
            pltpu.VMEM((QROWS, CHUNK), jnp.int32),
            pltpu.VMEM((QROWS, CHUNK), jnp.int32),
            pltpu.VMEM((CHUNK, H), jnp.float32),
            pltpu.VMEM((CHUNK, H), jnp.float32),
            pltpu.VMEM_SHARED((NPAD, H), jnp.float32),
            pltpu.SemaphoreType.DMA,
            pltpu.SemaphoreType.DMA,
            pltpu.SemaphoreType.DMA,
            pltpu.SemaphoreType.DMA,
        ],
    )(_sc_scatter_body)


def _sc_scatter_body(ya_hbm, yc_hbm, yd_hbm, src_hbm, dst_hbm, oa_hbm, oc_hbm,
                     od_hbm, src_v, dst_v, rows0, rows1, acc, gsem0, gsem1,
                     ssem0, ssem1):
    c = lax.axis_index("c")
    s = lax.axis_index("s")
    bufs = (rows0, rows1)
    gsems = (gsem0, gsem1)
    ssems = (ssem0, ssem1)
    ys = (ya_hbm, yc_hbm, yd_hbm)
    outs = (oa_hbm, oc_hbm, od_hbm)

    def gwait(y_hbm, b):
        pltpu.make_async_copy(y_hbm.at[pl.ds(0, CHUNK)], bufs[b],
                              gsems[b]).wait()

    def swait(y_hbm, b):
        pltpu.make_async_copy(y_hbm.at[pl.ds(0, CHUNK)], bufs[b],
                              ssems[b]).wait()

    base = c * (NROW // 2) + s * ROWS_W
    for t in range(3):
        y_hbm = ys[t]
        src_t = src_hbm.at[t]
        dst_t = dst_hbm.at[t]
        _memset2d(rows0, CHUNK)
        for b in range(ZROWS // CHUNK):
            pltpu.sync_copy(rows0,
                            acc.at[pl.ds(s * ZROWS + b * CHUNK, CHUNK)])
        plsc.subcore_barrier()

        for q in range(ROWS_W // QROWS):
            pltpu.sync_copy(src_t.at[pl.ds(base + q * QROWS, QROWS)], src_v)
            pltpu.sync_copy(dst_t.at[pl.ds(base + q * QROWS, QROWS)], dst_v)
            pltpu.async_copy(y_hbm.at[src_v.at[0]], rows0, gsem0)

            def pair(g, carry):
                for b in range(2):
                    j = g * 2 + b
                    gwait(y_hbm, b)
                    pltpu.async_copy(bufs[b], acc.at[dst_v.at[j]], ssems[b],
                                     add=True)

                    @pl.when(j >= 1)
                    def _():
                        swait(y_hbm, 1 - b)

                    @pl.when(j + 1 < QROWS)
                    def _():
                        pltpu.async_copy(y_hbm.at[src_v.at[j + 1]],
                                         bufs[1 - b], gsems[1 - b])

                return carry

            lax.fori_loop(0, QROWS // 2, pair, 0)
            swait(y_hbm, 1)
        plsc.subcore_barrier()
        pltpu.sync_copy(acc.at[pl.ds(s * ZROWS, ZROWS)],
                        outs[t].at[c].at[pl.ds(s * ZROWS, ZROWS)])
        if t < 2:
            plsc.subcore_barrier()


def _mm_body(h_ref, w_ref, cnt_ref, ya_ref, yc_ref, yd_ref):
    xw = jnp.dot(h_ref[...], w_ref[...], preferred_element_type=jnp.float32)
    cnt = cnt_ref[...]
    dinv = lax.rsqrt(cnt[0] + cnt[1] + 1.0)
    ya_ref[...] = xw[:, 0 * H:1 * H] * dinv[0][:, None]
    yc_ref[...] = xw[:, 1 * H:2 * H] * dinv[1][:, None]
    yd_ref[...] = xw[:, 2 * H:3 * H] * dinv[2][:, None]


def _tc_matmul(h, wcat, cnt):
    blk = 1024
    g = NPAD // blk
    return pl.pallas_call(
        _mm_body,
        grid=(g,),
        in_specs=[
            pl.BlockSpec((blk, H), lambda i: (i, 0)),
            pl.BlockSpec((H, 3 * H), lambda i: (0, 0)),
            pl.BlockSpec((NCORE, 3, blk), lambda i: (0, 0, i)),
        ],
        out_specs=[pl.BlockSpec((blk, H), lambda i: (i, 0))] * 3,
        out_shape=[jax.ShapeDtypeStruct((NPAD, H), jnp.float32)] * 3,
    )(h, wcat, cnt)


def _gru_body(sa_ref, sc_ref, sd_ref, ya_ref, yc_ref, yd_ref, h_ref, cnt_ref,
              bcat_ref, wih_ref, whh_ref, bih_ref, bhh_ref, out_ref):
    cnt = cnt_ref[...]
    dinv = lax.rsqrt(cnt[0] + cnt[1] + 1.0)
    bcat = bcat_ref[...]
    a = jnp.zeros_like(h_ref[...])
    for t, (s_ref, y_ref) in enumerate(((sa_ref, ya_ref), (sc_ref, yc_ref),
                                        (sd_ref, yd_ref))):
        st = s_ref[...]
        tot = st[0] + st[1] + y_ref[...]
        a = a + tot * dinv[t][:, None] + bcat[t][None, :]
    h = h_ref[...]
    gi = jnp.dot(a, wih_ref[...], preferred_element_type=jnp.float32) + bih_ref[...]
    gh = jnp.dot(h, whh_ref[...], preferred_element_type=jnp.float32) + bhh_ref[...]
    r = 1.0 / (1.0 + jnp.exp(-(gi[:, 0 * H:1 * H] + gh[:, 0 * H:1 * H])))
    z = 1.0 / (1.0 + jnp.exp(-(gi[:, 1 * H:2 * H] + gh[:, 1 * H:2 * H])))
    n = jnp.tanh(gi[:, 2 * H:3 * H] + r * gh[:, 2 * H:3 * H])
    out_ref[...] = (1.0 - z) * n + z * h


def _tc_gru(sa, sc, sd, ya, yc, yd, h, cnt, bcat, wihT, whhT, bih, bhh):
    blk = 1024
    g = NPAD // blk
    sspec = pl.BlockSpec((NCORE, blk, H), lambda i: (0, i, 0))
    yspec = pl.BlockSpec((blk, H), lambda i: (i, 0))
    return pl.pallas_call(
        _gru_body,
        grid=(g,),
        in_specs=[
            sspec, sspec, sspec, yspec, yspec, yspec, yspec,
            pl.BlockSpec((NCORE, 3, blk), lambda i: (0, 0, i)),
            pl.BlockSpec((3, H), lambda i: (0, 0)),
            pl.BlockSpec((H, 3 * H), lambda i: (0, 0)),
            pl.BlockSpec((H, 3 * H), lambda i: (0, 0)),
            pl.BlockSpec((1, 3 * H), lambda i: (0, 0)),
            pl.BlockSpec((1, 3 * H), lambda i: (0, 0)),
        ],
        out_specs=pl.BlockSpec((blk, H), lambda i: (i, 0)),
        out_shape=jax.ShapeDtypeStruct((NPAD, H), jnp.float32),
    )(sa, sc, sd, ya, yc, yd, h, cnt, bcat, wihT, whhT, bih, bhh)


def _prep_edges(e):
    e = e.astype(jnp.int32)
    pad = jnp.full((EPAD - E,), N, jnp.int32)
    src = jnp.concatenate([e[0], pad]).reshape(NROW, CHUNK)
    dst = jnp.concatenate([e[1], pad]).reshape(NROW, CHUNK)
    return src, dst


def kernel(x, edge_ast, edge_cfg, edge_dfg, W_ast, b_ast, W_cfg, b_cfg,
           W_dfg, b_dfg, W_ih, W_hh, b_ih, b_hh):
    src_a, dst_a = _prep_edges(edge_ast)
    src_c, dst_c = _prep_edges(edge_cfg)
    src_d, dst_d = _prep_edges(edge_dfg)
    src3 = jnp.stack([src_a, src_c, src_d])
    dst3 = jnp.stack([dst_a, dst_c, dst_d])

    h = jnp.zeros((NPAD, H), jnp.float32).at[:N].set(x)
    wcat = jnp.concatenate([W_ast, W_cfg, W_dfg], axis=1)
    bcat = jnp.stack([b_ast, b_cfg, b_dfg])
    wihT = W_ih.T
    whhT = W_hh.T
    bih = b_ih.reshape(1, 3 * H)
    bhh = b_hh.reshape(1, 3 * H)

    cnt = _make_sc_degree()(dst3).reshape(NCORE, 3, NPAD)

    scatter = _make_sc_scatter()
    for _ in range(L):
        ya, yc, yd = _tc_matmul(h, wcat, cnt)
        sa, sc, sd = scatter(ya, yc, yd, src3, dst3)
        h = _tc_gru(sa, sc, sd, ya, yc, yd, h, cnt, bcat, wihT, whhT, bih, bhh)
    return h[:N]


# speedup vs baseline: 1.0489x; 1.0093x over previous
import functools

import jax
import jax.numpy as jnp
from jax import lax
from jax.experimental import pallas as pl
from jax.experimental.pallas import tpu as pltpu
from jax.experimental.pallas import tpu_sc as plsc

N = 10000
H = 128
E = 320000
L = 3

NPAD = 10240
EPAD = 327680
CHUNK = 128
NROW = EPAD // CHUNK
NTILE = 16
NCORE = 2
ROWS_W = NROW // (NTILE * NCORE)
QROWS = 40
ZROWS = NPAD // NTILE

@functools.lru_cache(maxsize=None)
def _mesh():
    return plsc.VectorSubcoreMesh(core_axis_name="c", subcore_axis_name="s")


def _memset2d(ref, nrow):
    z16 = jnp.zeros((16,), jnp.float32)

    def body(k, carry):
        ref[k // 8, pl.ds((k % 8) * 16, 16)] = z16
        return carry

    lax.fori_loop(0, nrow * 8, body, 0)


def _memset1d(ref, n):
    z16 = jnp.zeros((16,), jnp.float32)

    def body(k, carry):
        ref[pl.ds(k * 16, 16)] = z16
        return carry

    lax.fori_loop(0, n // 16, body, 0)


@functools.lru_cache(maxsize=None)
def _make_sc_degree():
    return functools.partial(
        pl.kernel,
        mesh=_mesh(),
        out_type=jax.ShapeDtypeStruct((NCORE * 3 * NPAD,), jnp.float32),
        scratch_types=[
            pltpu.VMEM((ROWS_W, CHUNK), jnp.int32),
            pltpu.VMEM((CHUNK,), jnp.float32),
            pltpu.VMEM((ZROWS,), jnp.float32),
            pltpu.VMEM_SHARED((NPAD,), jnp.float32),
            pltpu.VMEM_SHARED((NPAD,), jnp.float32),
            pltpu.VMEM_SHARED((NPAD,), jnp.float32),
        ],
    )(_sc_degree_body)


def _sc_degree_body(dst_hbm, out_hbm, dst_v, ones_v, zvec_v, acc0, acc1, acc2):
    c = lax.axis_index("c")
    s = lax.axis_index("s")
    accs = (acc0, acc1, acc2)

    def o16(k, carry):
        ones_v[pl.ds(k * 16, 16)] = jnp.ones((16,), jnp.float32)
        return carry

    lax.fori_loop(0, CHUNK // 16, o16, 0)
    _memset1d(zvec_v, ZROWS)
    for t in range(3):
        pltpu.sync_copy(zvec_v, accs[t].at[pl.ds(s * ZROWS, ZROWS)])
    plsc.subcore_barrier()

    base = c * (NROW // 2) + s * ROWS_W
    for t in range(3):
        pltpu.sync_copy(dst_hbm.at[t].at[pl.ds(base, ROWS_W)], dst_v)

        def body(j, carry):
            pltpu.sync_copy(ones_v, accs[t].at[dst_v.at[j]], add=True)
            return carry

        lax.fori_loop(0, ROWS_W, body, 0)
    plsc.subcore_barrier()
    for t in range(3):
        off = (c * 3 + t) * NPAD + s * ZROWS
        pltpu.sync_copy(accs[t].at[pl.ds(s * ZROWS, ZROWS)],
                        out_hbm.at[pl.ds(off, ZROWS)])


@functools.lru_cache(maxsize=None)
def _make_sc_scatter():
    return functools.partial(
        pl.kernel,
        mesh=_mesh(),
        out_type=[jax.ShapeDtypeStruct((NCORE, NPAD, H), jnp.float32)] * 3,
        scratch_types=[
            pltpu.VMEM((QROWS, CHUNK), jnp.int32),
            pltpu.VMEM((QROWS, CHUNK), jnp.int32),
            pltpu.VMEM((CHUNK, H), jnp.float32),
            pltpu.VMEM((CHUNK, H), jnp.float32),
            pltpu.VMEM_SHARED((NPAD, H), jnp.float32),
            pltpu.SemaphoreType.DMA,
            pltpu.SemaphoreType.DMA,
            pltpu.SemaphoreType.DMA,
            pltpu.SemaphoreType.DMA,
        ],
    )(_sc_scatter_body)


def _sc_scatter_body(ya_hbm, yc_hbm, yd_hbm, src_hbm, dst_hbm, oa_hbm, oc_hbm,
                     od_hbm, src_v, dst_v, rows0, rows1, acc, gsem0, gsem1,
                     ssem0, ssem1):
    c = lax.axis_index("c")
    s = lax.axis_index("s")
    bufs = (rows0, rows1)
    gsems = (gsem0, gsem1)
    ssems = (ssem0, ssem1)
    ys = (ya_hbm, yc_hbm, yd_hbm)
    outs = (oa_hbm, oc_hbm, od_hbm)

    def gwait(y_hbm, b):
        pltpu.make_async_copy(y_hbm.at[pl.ds(0, CHUNK)], bufs[b],
                              gsems[b]).wait()

    def swait(y_hbm, b):
        pltpu.make_async_copy(y_hbm.at[pl.ds(0, CHUNK)], bufs[b],
                              ssems[b]).wait()

    base = c * (NROW // 2) + s * ROWS_W
    for t in range(3):
        y_hbm = ys[t]
        src_t = src_hbm.at[t]
        dst_t = dst_hbm.at[t]
        _memset2d(rows0, CHUNK)
        for b in range(ZROWS // CHUNK):
            pltpu.sync_copy(rows0,
                            acc.at[pl.ds(s * ZROWS + b * CHUNK, CHUNK)])
        plsc.subcore_barrier()

        for q in range(ROWS_W // QROWS):
            pltpu.sync_copy(src_t.at[pl.ds(base + q * QROWS, QROWS)], src_v)
            pltpu.sync_copy(dst_t.at[pl.ds(base + q * QROWS, QROWS)], dst_v)
            pltpu.async_copy(y_hbm.at[src_v.at[0]], rows0, gsem0)

            def pair(g, carry):
                for b in range(2):
                    j = g * 2 + b
                    gwait(y_hbm, b)
                    pltpu.async_copy(bufs[b], acc.at[dst_v.at[j]], ssems[b],
                                     add=True)

                    @pl.when(j >= 1)
                    def _():
                        swait(y_hbm, 1 - b)

                    @pl.when(j + 1 < QROWS)
                    def _():
                        pltpu.async_copy(y_hbm.at[src_v.at[j + 1]],
                                         bufs[1 - b], gsems[1 - b])

                return carry

            lax.fori_loop(0, QROWS // 2, pair, 0)
            swait(y_hbm, 1)
        plsc.subcore_barrier()
        pltpu.sync_copy(acc.at[pl.ds(s * ZROWS, ZROWS)],
                        outs[t].at[c].at[pl.ds(s * ZROWS, ZROWS)])
        if t < 2:
            plsc.subcore_barrier()


def _mm_body(h_ref, w_ref, cnt_ref, ya_ref, yc_ref, yd_ref):
    xw = jnp.dot(h_ref[...], w_ref[...], preferred_element_type=jnp.float32)
    cnt = cnt_ref[...]
    dinv = lax.rsqrt(cnt[0] + cnt[1] + 1.0)
    ya_ref[...] = xw[:, 0 * H:1 * H] * dinv[0][:, None]
    yc_ref[...] = xw[:, 1 * H:2 * H] * dinv[1][:, None]
    yd_ref[...] = xw[:, 2 * H:3 * H] * dinv[2][:, None]


def _tc_matmul(h, wcat, cnt):
    blk = 1024
    g = NPAD // blk
    return pl.pallas_call(
        _mm_body,
        grid=(g,),
        in_specs=[
            pl.BlockSpec((blk, H), lambda i: (i, 0)),
            pl.BlockSpec((H, 3 * H), lambda i: (0, 0)),
            pl.BlockSpec((NCORE, 3, blk), lambda i: (0, 0, i)),
        ],
        out_specs=[pl.BlockSpec((blk, H), lambda i: (i, 0))] * 3,
        out_shape=[jax.ShapeDtypeStruct((NPAD, H), jnp.float32)] * 3,
    )(h, wcat, cnt)


def _gru_body(sa_ref, sc_ref, sd_ref, ya_ref, yc_ref, yd_ref, h_ref, cnt_ref,
              bcat_ref, wih_ref, whh_ref, bih_ref, bhh_ref, out_ref):
    cnt = cnt_ref[...]
    dinv = lax.rsqrt(cnt[0] + cnt[1] + 1.0)
    bcat = bcat_ref[...]
    a = jnp.zeros_like(h_ref[...])
    for t, (s_ref, y_ref) in enumerate(((sa_ref, ya_ref), (sc_ref, yc_ref),
                                        (sd_ref, yd_ref))):
        st = s_ref[...]
        tot = st[0] + st[1] + y_ref[...]
        a = a + tot * dinv[t][:, None] + bcat[t][None, :]
    h = h_ref[...]
    gi = jnp.dot(a, wih_ref[...], preferred_element_type=jnp.float32) + bih_ref[...]
    gh = jnp.dot(h, whh_ref[...], preferred_element_type=jnp.float32) + bhh_ref[...]
    r = 1.0 / (1.0 + jnp.exp(-(gi[:, 0 * H:1 * H] + gh[:, 0 * H:1 * H])))
    z = 1.0 / (1.0 + jnp.exp(-(gi[:, 1 * H:2 * H] + gh[:, 1 * H:2 * H])))
    n = jnp.tanh(gi[:, 2 * H:3 * H] + r * gh[:, 2 * H:3 * H])
    out_ref[...] = (1.0 - z) * n + z * h


def _tc_gru(sa, sc, sd, ya, yc, yd, h, cnt, bcat, wihT, whhT, bih, bhh):
    blk = 1024
    g = NPAD // blk
    sspec = pl.BlockSpec((NCORE, blk, H), lambda i: (0, i, 0))
    yspec = pl.BlockSpec((blk, H), lambda i: (i, 0))
    return pl.pallas_call(
        _gru_body,
        grid=(g,),
        in_specs=[
            sspec, sspec, sspec, yspec, yspec, yspec, yspec,
            pl.BlockSpec((NCORE, 3, blk), lambda i: (0, 0, i)),
            pl.BlockSpec((3, H), lambda i: (0, 0)),
            pl.BlockSpec((H, 3 * H), lambda i: (0, 0)),
            pl.BlockSpec((H, 3 * H), lambda i: (0, 0)),
            pl.BlockSpec((1, 3 * H), lambda i: (0, 0)),
            pl.BlockSpec((1, 3 * H), lambda i: (0, 0)),
        ],
        out_specs=pl.BlockSpec((blk, H), lambda i: (i, 0)),
        out_shape=jax.ShapeDtypeStruct((NPAD, H), jnp.float32),
    )(sa, sc, sd, ya, yc, yd, h, cnt, bcat, wihT, whhT, bih, bhh)


def _prep_edges(e):
    e = e.astype(jnp.int32)
    pad = jnp.full((EPAD - E,), N, jnp.int32)
    src = jnp.concatenate([e[0], pad]).reshape(NROW, CHUNK)
    dst = jnp.concatenate([e[1], pad]).reshape(NROW, CHUNK)
    return src, dst


def kernel(x, edge_ast, edge_cfg, edge_dfg, W_ast, b_ast, W_cfg, b_cfg,
           W_dfg, b_dfg, W_ih, W_hh, b_ih, b_hh):
    src_a, dst_a = _prep_edges(edge_ast)
    src_c, dst_c = _prep_edges(edge_cfg)
    src_d, dst_d = _prep_edges(edge_dfg)
    src3 = jnp.stack([src_a, src_c, src_d])
    dst3 = jnp.stack([dst_a, dst_c, dst_d])

    h = jnp.zeros((NPAD, H), jnp.float32).at[:N].set(x)
    wcat = jnp.concatenate([W_ast, W_cfg, W_dfg], axis=1)
    bcat = jnp.stack([b_ast, b_cfg, b_dfg])
    wihT = W_ih.T
    whhT = W_hh.T
    bih = b_ih.reshape(1, 3 * H)
    bhh = b_hh.reshape(1, 3 * H)

    cnt = _make_sc_degree()(dst3).reshape(NCORE, 3, NPAD)

    scatter = _make_sc_scatter()
    for _ in range(L):
        ya, yc, yd = _tc_matmul(h, wcat, cnt)
        sa, sc, sd = scatter(ya, yc, yd, src3, dst3)
        h = _tc_gru(sa, sc, sd, ya, yc, yd, h, cnt, bcat, wihT, whhT, bih, bhh)
    return h[:N]
